# baseline (speedup 1.0000x reference)
import functools

import jax
import jax.numpy as jnp
from jax import lax
from jax.experimental import pallas as pl
from jax.experimental.pallas import tpu as pltpu
from jax.experimental.pallas import tpu_sc as plsc

HIDDEN = 128
BATCH = 16384
NC = 2
NS = 16
NW = NC * NS

PER_W = BATCH // NW
CH = 256
NCH = PER_W // CH

REL_PAD = 512

_EPS = 1e-12


def _sc_gather(node_emb, head_idx, tail_idx):
    mesh = plsc.VectorSubcoreMesh(core_axis_name="c", subcore_axis_name="s")
    out = jax.ShapeDtypeStruct((BATCH, HIDDEN), jnp.float32)

    @functools.partial(
        pl.kernel,
        mesh=mesh,
        out_type=[out, out],
        scratch_types=[
            pltpu.VMEM((PER_W,), jnp.int32),
            pltpu.VMEM((PER_W,), jnp.int32),
            pltpu.VMEM((CH, HIDDEN), jnp.float32),
            pltpu.VMEM((CH, HIDDEN), jnp.float32),
            pltpu.SemaphoreType.DMA,
            pltpu.SemaphoreType.DMA,
        ],
    )
    def gather_kernel(node_hbm, hidx_hbm, tidx_hbm, h_out, t_out,
                      hidx_v, tidx_v, rows0, rows1, s0, s1):
        wid = lax.axis_index("s") * NC + lax.axis_index("c")
        base = wid * PER_W

        pltpu.sync_copy(hidx_hbm.at[pl.ds(base, PER_W)], hidx_v)

        rows = [rows0, rows1]
        sems = [s0, s1]
        chunks = []
        for idx_v, out_ref in ((hidx_v, h_out), (tidx_v, t_out)):
            for j in range(NCH):
                chunks.append((idx_v, j * CH, out_ref))
        n = len(chunks)

        def start(k):
            idx_v, off, _ = chunks[k]
            return pltpu.async_copy(
                node_hbm.at[idx_v.at[pl.ds(off, CH)]], rows[k % 2],
                sems[k % 2])

        inflight = {0: start(0), 1: start(1)}
        pltpu.sync_copy(tidx_hbm.at[pl.ds(base, PER_W)], tidx_v)
        for k in range(n):
            inflight[k].wait()
            _, off, out_ref = chunks[k]
            pltpu.sync_copy(rows[k % 2], out_ref.at[pl.ds(base + off, CH)])
            if k + 2 < n:
                inflight[k + 2] = start(k + 2)

    return gather_kernel(node_emb, head_idx, tail_idx)


_REL_BLK = 2048


def _rel_body(idx_ref, rel_ref, o_ref):
    idx = idx_ref[0, :]
    one_hot = (lax.broadcasted_iota(jnp.int32, (_REL_BLK, REL_PAD), 1)
               == idx[:, None]).astype(jnp.bfloat16)
    x = rel_ref[...]
    hi = x.astype(jnp.bfloat16)
    lo = (x - hi.astype(jnp.float32)).astype(jnp.bfloat16)
    acc = jnp.dot(one_hot, hi, preferred_element_type=jnp.float32)
    acc = acc + jnp.dot(one_hot, lo, preferred_element_type=jnp.float32)
    o_ref[...] = acc


def _tc_rel_gather(rel_emb_pad, rel_idx):
    return pl.pallas_call(
        _rel_body,
        grid=(BATCH // _REL_BLK,),
        in_specs=[
            pl.BlockSpec((1, _REL_BLK), lambda i: (0, i)),
            pl.BlockSpec((REL_PAD, HIDDEN), lambda i: (0, 0)),
        ],
        out_specs=pl.BlockSpec((_REL_BLK, HIDDEN), lambda i: (i, 0)),
        out_shape=jax.ShapeDtypeStruct((BATCH, HIDDEN), jnp.float32),
    )(rel_idx.reshape(1, BATCH), rel_emb_pad)


def _norm_body(h_ref, t_ref, oh_ref, ot_ref):
    for src, dst in ((h_ref, oh_ref), (t_ref, ot_ref)):
        x = src[...]
        nrm = jnp.sum(jnp.abs(x), axis=-1, keepdims=True)
        dst[...] = x * (1.0 / jnp.maximum(nrm, _EPS))


def _tc_normalize(h, t):
    blk = 8192
    spec = pl.BlockSpec((blk, HIDDEN), lambda i: (i, 0))
    out = jax.ShapeDtypeStruct((BATCH, HIDDEN), jnp.float32)
    return pl.pallas_call(
        _norm_body,
        grid=(BATCH // blk,),
        in_specs=[spec, spec],
        out_specs=[spec, spec],
        out_shape=[out, out],
        input_output_aliases={0: 0, 1: 1},
    )(h, t)


@jax.jit
def kernel(head_index, rel_type, tail_index, node_emb, rel_emb):
    rel_emb_pad = jnp.pad(rel_emb, ((0, REL_PAD - rel_emb.shape[0]), (0, 0)))
    h_raw, t_raw = _sc_gather(node_emb,
                              head_index.astype(jnp.int32),
                              tail_index.astype(jnp.int32))
    rel = _tc_rel_gather(rel_emb_pad, rel_type.astype(jnp.int32))
    head, tail = _tc_normalize(h_raw, t_raw)
    return (head, rel, tail)


# speedup vs baseline: 1.0206x; 1.0206x over previous
import functools

import jax
import jax.numpy as jnp
from jax import lax
from jax.experimental import pallas as pl
from jax.experimental.pallas import tpu as pltpu
from jax.experimental.pallas import tpu_sc as plsc

HIDDEN = 128
BATCH = 16384
NC = 2
NS = 16
NW = NC * NS

PER_W = BATCH // NW
CH = 256
NCH = PER_W // CH

REL_PAD = 512

_EPS = 1e-12


def _sc_gather(node_emb, head_idx, tail_idx):
    mesh = plsc.VectorSubcoreMesh(core_axis_name="c", subcore_axis_name="s")
    out = jax.ShapeDtypeStruct((BATCH, HIDDEN), jnp.float32)

    @functools.partial(
        pl.kernel,
        mesh=mesh,
        out_type=[out, out],
        scratch_types=[
            pltpu.VMEM((PER_W,), jnp.int32),
            pltpu.VMEM((PER_W,), jnp.int32),
            pltpu.VMEM((CH, HIDDEN), jnp.float32),
            pltpu.VMEM((CH, HIDDEN), jnp.float32),
            pltpu.SemaphoreType.DMA,
            pltpu.SemaphoreType.DMA,
        ],
    )
    def gather_kernel(node_hbm, hidx_hbm, tidx_hbm, h_out, t_out,
                      hidx_v, tidx_v, rows0, rows1, s0, s1):
        wid = lax.axis_index("s") * NC + lax.axis_index("c")
        base = wid * PER_W

        pltpu.sync_copy(hidx_hbm.at[pl.ds(base, PER_W)], hidx_v)

        rows = [rows0, rows1]
        sems = [s0, s1]
        chunks = []
        for idx_v, out_ref in ((hidx_v, h_out), (tidx_v, t_out)):
            for j in range(NCH):
                chunks.append((idx_v, j * CH, out_ref))
        n = len(chunks)

        def start(k):
            idx_v, off, _ = chunks[k]
            return pltpu.async_copy(
                node_hbm.at[idx_v.at[pl.ds(off, CH)]], rows[k % 2],
                sems[k % 2])

        inflight = {0: start(0), 1: start(1)}
        pltpu.sync_copy(tidx_hbm.at[pl.ds(base, PER_W)], tidx_v)
        for k in range(n):
            inflight[k].wait()
            _, off, out_ref = chunks[k]
            pltpu.sync_copy(rows[k % 2], out_ref.at[pl.ds(base + off, CH)])
            if k + 2 < n:
                inflight[k + 2] = start(k + 2)

    return gather_kernel(node_emb, head_idx, tail_idx)


_REL_BLK = 2048


def _rel_body(idx_ref, rel_ref, o_ref):
    idx = idx_ref[0, :]
    one_hot = (lax.broadcasted_iota(jnp.int32, (_REL_BLK, REL_PAD), 1)
               == idx[:, None]).astype(jnp.bfloat16)
    x = rel_ref[...]
    hi = x.astype(jnp.bfloat16)
    lo = (x - hi.astype(jnp.float32)).astype(jnp.bfloat16)
    acc = jnp.dot(one_hot, hi, preferred_element_type=jnp.float32)
    acc = acc + jnp.dot(one_hot, lo, preferred_element_type=jnp.float32)
    o_ref[...] = acc


def _tc_rel_gather(rel_emb_pad, rel_idx):
    return pl.pallas_call(
        _rel_body,
        grid=(BATCH // _REL_BLK,),
        in_specs=[
            pl.BlockSpec((1, _REL_BLK), lambda i: (0, i)),
            pl.BlockSpec((REL_PAD, HIDDEN), lambda i: (0, 0)),
        ],
        out_specs=pl.BlockSpec((_REL_BLK, HIDDEN), lambda i: (i, 0)),
        out_shape=jax.ShapeDtypeStruct((BATCH, HIDDEN), jnp.float32),
    )(rel_idx.reshape(1, BATCH), rel_emb_pad)


def _norm_body(h_ref, t_ref, oh_ref, ot_ref):
    for src, dst in ((h_ref, oh_ref), (t_ref, ot_ref)):
        x = src[...]
        nrm = jnp.sum(jnp.abs(x), axis=-1, keepdims=True)
        dst[...] = x * (1.0 / jnp.maximum(nrm, _EPS))


def _tc_normalize(h, t):
    blk = 8192
    spec = pl.BlockSpec((blk, HIDDEN), lambda i: (i, 0))
    out = jax.ShapeDtypeStruct((BATCH, HIDDEN), jnp.float32)
    return pl.pallas_call(
        _norm_body,
        grid=(BATCH // blk,),
        in_specs=[spec, spec],
        out_specs=[spec, spec],
        out_shape=[out, out],
    )(h, t)


@jax.jit
def kernel(head_index, rel_type, tail_index, node_emb, rel_emb):
    rel_emb_pad = jnp.pad(rel_emb, ((0, REL_PAD - rel_emb.shape[0]), (0, 0)))
    h_raw, t_raw = _sc_gather(node_emb,
                              head_index.astype(jnp.int32),
                              tail_index.astype(jnp.int32))
    rel = _tc_rel_gather(rel_emb_pad, rel_type.astype(jnp.int32))
    head, tail = _tc_normalize(h_raw, t_raw)
    return (head, rel, tail)
